# Initial kernel scaffold; baseline (speedup 1.0000x reference)
import jax
import jax.numpy as jnp
from jax.experimental import pallas as pl


def kernel(mem, idx, val):
    raise NotImplementedError("write your pallas kernel here")


# speedup vs baseline: 5.5287x; 5.5287x over previous
import functools

import jax
import jax.numpy as jnp
from jax import lax
from jax.experimental import pallas as pl
from jax.experimental.pallas import tpu as pltpu
from jax.experimental.pallas import tpu_sc as plsc

NCLASS = 19
MEMORY_SIZE = 2000
FEAT_DIM = 256
INV_TEMP = 10.0
B = 131072
M = NCLASS * MEMORY_SIZE

NC, NS, L = 2, 16, 16
NW = NC * NS
M_PAD = 38912
MS = M_PAD // NS
CH = B // NW
GROUPS = CH // L
MGROUPS = MS // L

_SC_MESH = plsc.VectorSubcoreMesh(core_axis_name="c", subcore_axis_name="s",
                                  num_cores=NC, num_subcores=NS)


@functools.partial(
    pl.kernel,
    out_type=jax.ShapeDtypeStruct((NC * M_PAD,), jnp.int32),
    mesh=_SC_MESH,
    scratch_types=[
        pltpu.VMEM((CH,), jnp.int32),
        pltpu.VMEM((M_PAD,), jnp.int32),
        pltpu.VMEM_SHARED((NS * M_PAD,), jnp.int32),
        pltpu.VMEM((NS * MS,), jnp.int32),
        pltpu.VMEM((MS,), jnp.int32),
        pltpu.SemaphoreType.DMA,
    ],
    compiler_params=pltpu.CompilerParams(needs_layout_passes=False),
)
def _winner_tables(idx_hbm, w01_hbm, idx_v, wp_v, shared, merge_v, acc_v, sem):
    c = lax.axis_index("c")
    s = lax.axis_index("s")
    base = c * (B // NC) + s * CH

    pltpu.sync_copy(idx_hbm.at[pl.ds(base, CH)], idx_v)

    neg1 = jnp.full((L,), -1, jnp.int32)

    def _init(j, _):
        off = pl.multiple_of(j * (4 * L), 4 * L)
        for u in range(4):
            wp_v[pl.ds(off + u * L, L)] = neg1
        return _
    lax.fori_loop(0, M_PAD // (4 * L), _init, None)

    lane = lax.iota(jnp.int32, L)

    def _scatter(g, _):
        off = pl.multiple_of(g * L, L)
        vidx = idx_v[pl.ds(off, L)]
        ival = base + g * L + lane
        plsc.store_scatter(wp_v, [vidx], ival)
        for _r in range(2):
            cur = plsc.load_gather(wp_v, [vidx])
            plsc.store_scatter(wp_v, [vidx], ival, mask=ival > cur)
        return _
    lax.fori_loop(0, GROUPS, _scatter, None)

    pltpu.sync_copy(wp_v, shared.at[pl.ds(s * M_PAD, M_PAD)])
    plsc.subcore_barrier()

    handles = [
        pltpu.async_copy(shared.at[pl.ds(r * M_PAD + s * MS, MS)],
                         merge_v.at[pl.ds(r * MS, MS)], sem)
        for r in range(NS)
    ]
    for h in handles:
        h.wait()

    def _merge(g, _):
        off = pl.multiple_of(g * L, L)
        v = merge_v[pl.ds(off, L)]
        for r in range(1, NS):
            v = jnp.maximum(v, merge_v[pl.ds(r * MS + off, L)])
        acc_v[pl.ds(off, L)] = v
        return _
    lax.fori_loop(0, MGROUPS, _merge, None)

    pltpu.sync_copy(acc_v, w01_hbm.at[pl.ds(c * M_PAD + s * MS, MS)])


@functools.partial(
    pl.kernel,
    out_type=jax.ShapeDtypeStruct((B,), jnp.float32),
    mesh=_SC_MESH,
    scratch_types=[
        pltpu.VMEM((MS,), jnp.int32),
        pltpu.VMEM((MS,), jnp.int32),
        pltpu.VMEM_SHARED((M_PAD,), jnp.int32),
        pltpu.VMEM((M_PAD,), jnp.int32),
        pltpu.VMEM((CH,), jnp.int32),
        pltpu.VMEM((CH,), jnp.float32),
        pltpu.SemaphoreType.DMA,
    ],
    compiler_params=pltpu.CompilerParams(needs_layout_passes=False),
)
def _flags_and_mask(w01_hbm, idx_hbm, flags_hbm,
                    w0_v, w1_v, wm_sh, wfull_v, idx_v, flags_v, sem):
    c = lax.axis_index("c")
    s = lax.axis_index("s")

    h0 = pltpu.async_copy(w01_hbm.at[pl.ds(s * MS, MS)], w0_v, sem)
    h1 = pltpu.async_copy(w01_hbm.at[pl.ds(M_PAD + s * MS, MS)], w1_v, sem)
    h0.wait()
    h1.wait()

    def _merge(g, _):
        off = pl.multiple_of(g * L, L)
        w0_v[pl.ds(off, L)] = jnp.maximum(w0_v[pl.ds(off, L)],
                                          w1_v[pl.ds(off, L)])
        return _
    lax.fori_loop(0, MGROUPS, _merge, None)

    pltpu.sync_copy(w0_v, wm_sh.at[pl.ds(s * MS, MS)])
    plsc.subcore_barrier()
    pltpu.sync_copy(wm_sh, wfull_v)

    base = c * (B // NC) + s * CH
    pltpu.sync_copy(idx_hbm.at[pl.ds(base, CH)], idx_v)
    lane = lax.iota(jnp.int32, L)

    def _flag(g, _):
        off = pl.multiple_of(g * L, L)
        vidx = idx_v[pl.ds(off, L)]
        wv = plsc.load_gather(wfull_v, [vidx])
        ival = base + g * L + lane
        flags_v[pl.ds(off, L)] = jnp.where(wv == ival, 1.0, 0.0).astype(jnp.float32)
        return _
    lax.fori_loop(0, GROUPS, _flag, None)

    pltpu.sync_copy(flags_v, flags_hbm.at[pl.ds(base, CH)])


RB = 16384
NBLK = B // RB


def _surviving_mem_body(mem_ref, w0_ref, w1_ref, out_ref):
    cid = pl.program_id(0)
    mm = mem_ref[...]
    w = jnp.maximum(w0_ref[0, 0, :], w1_ref[0, 0, :])
    keep = jnp.where(w < 0, 1.0, 0.0).astype(jnp.float32)
    srow = jnp.sum(mm * keep[:, None], axis=0)
    out_ref[pl.ds(cid, 1), :] = srow[None, :]


def _fused_val_body(val_ref, idx_ref, flg_ref, smem_ref, out_ref, acc_ref):
    i = pl.program_id(0)
    v = val_ref[...]
    ones_row = jnp.ones((1, FEAT_DIM), jnp.float32)
    ss_t = lax.dot_general(ones_row, v * v, (((1,), (1,)), ((), ())),
                           preferred_element_type=jnp.float32)
    rn_t = 1.0 / (jnp.sqrt(ss_t) + 1e-12)
    labels = idx_ref[0, 0, :] // MEMORY_SIZE
    onehot_t = (labels[None, :] ==
                lax.broadcasted_iota(jnp.int32, (NCLASS, RB), 0)).astype(jnp.float32)

    @pl.when(i == 0)
    def _():
        acc_ref[...] = jnp.zeros_like(acc_ref)

    @pl.when(i < NBLK)
    def _():
        a_t = onehot_t * (flg_ref[0, 0, :][None, :] * rn_t)
        acc_ref[...] += lax.dot_general(a_t, v, (((1,), (0,)), ((), ())),
                                        preferred_element_type=jnp.float32)

    @pl.when(i >= NBLK)
    def _():
        mean = (smem_ref[...] + acc_ref[...]) * (1.0 / MEMORY_SIZE)
        nrm = jnp.sqrt(jnp.sum(mean * mean, axis=1, keepdims=True))
        mf = mean * (INV_TEMP / (nrm + 1e-12))
        logits_t = lax.dot_general(mf, v, (((1,), (1,)), ((), ())),
                                   preferred_element_type=jnp.float32) * rn_t
        pos = jnp.sum(logits_t * onehot_t, axis=0)
        mx = jnp.max(logits_t, axis=0)
        lse = mx + jnp.log(jnp.sum(jnp.exp(logits_t - mx[None, :]), axis=0))
        out_ref[0, 0, :] = lse - pos


def kernel(mem, idx, val):
    idx = idx.astype(jnp.int32)

    w01 = _winner_tables(idx)
    flags = _flags_and_mask(w01, idx)

    idx3 = idx.reshape(NBLK, 1, RB)
    flg3 = flags.reshape(NBLK, 1, RB)

    _rem = lambda i: (lax.rem(i, NBLK), 0)
    _rem3 = lambda i: (lax.rem(i, NBLK), 0, 0)

    w0_3 = w01[:M].reshape(NCLASS, 1, MEMORY_SIZE)
    w1_3 = w01[M_PAD:M_PAD + M].reshape(NCLASS, 1, MEMORY_SIZE)
    smem = pl.pallas_call(
        _surviving_mem_body,
        grid=(NCLASS,),
        in_specs=[
            pl.BlockSpec((MEMORY_SIZE, FEAT_DIM), lambda i: (i, 0)),
            pl.BlockSpec((1, 1, MEMORY_SIZE), lambda i: (i, 0, 0)),
            pl.BlockSpec((1, 1, MEMORY_SIZE), lambda i: (i, 0, 0)),
        ],
        out_specs=pl.BlockSpec((NCLASS, FEAT_DIM), lambda i: (0, 0)),
        out_shape=jax.ShapeDtypeStruct((NCLASS, FEAT_DIM), jnp.float32),
    )(mem, w0_3, w1_3)

    loss3 = pl.pallas_call(
        _fused_val_body,
        grid=(2 * NBLK,),
        in_specs=[
            pl.BlockSpec((RB, FEAT_DIM), _rem),
            pl.BlockSpec((1, 1, RB), _rem3),
            pl.BlockSpec((1, 1, RB), _rem3),
            pl.BlockSpec((NCLASS, FEAT_DIM), lambda i: (0, 0)),
        ],
        out_specs=pl.BlockSpec((1, 1, RB), _rem3),
        out_shape=jax.ShapeDtypeStruct((NBLK, 1, RB), jnp.float32),
        scratch_shapes=[pltpu.VMEM((NCLASS, FEAT_DIM), jnp.float32)],
    )(val, idx3, flg3, smem)

    return loss3.reshape(B)
